# baseline (speedup 1.0000x reference)
import jax
import jax.numpy as jnp
from jax import lax
from jax.experimental import pallas as pl
from jax.experimental.pallas import tpu as pltpu
from jax.experimental.pallas import tpu_sc as plsc

NUM_BINS = 100000
NC, NS, L = 2, 16, 16
NW = NC * NS

ROWS, COLS = 16384, 26
PADC = 128
DMAC = 128
RPW = ROWS // NW
UNROLL = 4
STEPS = RPW // UNROLL

_MB1 = 2684
_MB0 = 23237
_C1 = 0x7FEB352D
_C2 = 0x846CA68B - (1 << 32)


def _srl(x, k):
    return lax.shift_right_logical(x, jnp.int32(k))


def _hash_mod(x):
    x = x ^ _srl(x, 16)
    x = x * jnp.int32(_C1)
    x = x ^ _srl(x, 15)
    x = x * jnp.int32(_C2)
    h = x ^ _srl(x, 16)
    xs = _srl(h, 5)
    a1 = _srl(xs, 16)
    a0 = xs & jnp.int32(0xFFFF)
    t = _srl(a0 * jnp.int32(_MB0), 16)
    u = a1 * jnp.int32(_MB0) + t
    v = a0 * jnp.int32(_MB1) + (u & jnp.int32(0xFFFF))
    hi = a1 * jnp.int32(_MB1) + _srl(u, 16) + _srl(v, 16)
    q = _srl(hi, 7)
    return h - q * jnp.int32(NUM_BINS)


def _sc_body(in_hbm, out_hbm, buf):
    wid = lax.axis_index("s") * NC + lax.axis_index("c")
    base = wid * RPW
    pltpu.sync_copy(in_hbm.at[pl.ds(base, RPW), pl.ds(0, DMAC)], buf)

    def step(i, carry):
        r0 = i * UNROLL
        for u in range(UNROLL):
            r = r0 + u
            lo = (r, pl.ds(0, L))
            hi = (r, pl.ds(COLS - L, L))
            xlo = buf[lo]
            xhi = buf[hi]
            buf[lo] = _hash_mod(xlo)
            buf[hi] = _hash_mod(xhi)
        return carry

    lax.fori_loop(0, STEPS, step, 0)
    pltpu.sync_copy(buf, out_hbm.at[pl.ds(base, RPW), pl.ds(0, DMAC)])


@jax.jit
def kernel(inputs):
    padded = jnp.pad(inputs, ((0, 0), (0, PADC - COLS)))
    call = pl.kernel(
        _sc_body,
        out_type=jax.ShapeDtypeStruct((ROWS, PADC), jnp.int32),
        mesh=plsc.VectorSubcoreMesh(core_axis_name="c", subcore_axis_name="s"),
        scratch_types=[pltpu.VMEM((RPW, DMAC), jnp.int32)],
    )
    return call(padded)[:, :COLS]


# speedup vs baseline: 1.2980x; 1.2980x over previous
import jax
import jax.numpy as jnp
from jax import lax
from jax.experimental import pallas as pl
from jax.experimental.pallas import tpu as pltpu
from jax.experimental.pallas import tpu_sc as plsc

NUM_BINS = 100000
NC, NS, L = 2, 16, 16
NW = NC * NS

ROWS, COLS = 16384, 26
CPW = ROWS // NW
VSTEP = 2
STEPS = CPW // (L * VSTEP)

_MB1 = 2684
_MB0 = 23237
_C1 = 0x7FEB352D
_C2 = 0x846CA68B - (1 << 32)


def _srl(x, k):
    return lax.shift_right_logical(x, jnp.int32(k))


def _hash_mod(x):
    x = x ^ _srl(x, 16)
    x = x * jnp.int32(_C1)
    x = x ^ _srl(x, 15)
    x = x * jnp.int32(_C2)
    h = x ^ _srl(x, 16)
    xs = _srl(h, 5)
    a1 = _srl(xs, 16)
    a0 = xs & jnp.int32(0xFFFF)
    t = _srl(a0 * jnp.int32(_MB0), 16)
    u = a1 * jnp.int32(_MB0) + t
    v = a0 * jnp.int32(_MB1) + (u & jnp.int32(0xFFFF))
    hi = a1 * jnp.int32(_MB1) + _srl(u, 16) + _srl(v, 16)
    q = _srl(hi, 7)
    return h - q * jnp.int32(NUM_BINS)


def _sc_body(in_hbm, out_hbm, buf):
    wid = lax.axis_index("s") * NC + lax.axis_index("c")
    base = wid * CPW
    pltpu.sync_copy(in_hbm.at[:, pl.ds(base, CPW)], buf)

    def step(i, carry):
        c0 = i * (L * VSTEP)
        for r in range(COLS):
            for j in range(VSTEP):
                sl = (r, pl.ds(c0 + j * L, L))
                buf[sl] = _hash_mod(buf[sl])
        return carry

    lax.fori_loop(0, STEPS, step, 0)
    pltpu.sync_copy(buf, out_hbm.at[:, pl.ds(base, CPW)])


@jax.jit
def kernel(inputs):
    tin = inputs.T
    call = pl.kernel(
        _sc_body,
        out_type=jax.ShapeDtypeStruct((COLS, ROWS), jnp.int32),
        mesh=plsc.VectorSubcoreMesh(core_axis_name="c", subcore_axis_name="s"),
        scratch_types=[pltpu.VMEM((COLS, CPW), jnp.int32)],
    )
    return call(tin).T
